# baseline (speedup 1.0000x reference)
import functools

import jax
import jax.numpy as jnp
from jax import lax
from jax.experimental import pallas as pl
from jax.experimental.pallas import tpu as pltpu
from jax.experimental.pallas import tpu_sc as plsc

NC = 2
NS = 16
NW = NC * NS
C = 128
L = 16


def _sc_mesh():
    return plsc.VectorSubcoreMesh(core_axis_name="c", subcore_axis_name="s")


def _make_deg_kernel(R, NCH):
    @functools.partial(
        pl.kernel,
        out_type=jax.ShapeDtypeStruct((NC * R,), jnp.float32),
        mesh=_sc_mesh(),
        scratch_types=[
            pltpu.VMEM((NCH, C), jnp.int32),
            pltpu.VMEM((C,), jnp.float32),
            pltpu.VMEM((R // NS,), jnp.float32),
            pltpu.VMEM_SHARED((R,), jnp.float32),
        ],
    )
    def deg_kernel(dst_hbm, out_hbm, dall, ones_v, zv, deg_sh):
        cid = lax.axis_index("c")
        sid = lax.axis_index("s")
        wid = sid * NC + cid
        stripe = R // NS

        ones16 = jnp.ones((L,), jnp.float32)
        zero16 = jnp.zeros((L,), jnp.float32)
        for j in range(C // L):
            ones_v[pl.ds(j * L, L)] = ones16
        for j in range(stripe // L):
            zv[pl.ds(j * L, L)] = zero16
        pltpu.sync_copy(dst_hbm.at[wid], dall)
        pltpu.sync_copy(zv, deg_sh.at[pl.ds(sid * stripe, stripe)])
        plsc.subcore_barrier()

        def body(k, _):
            pltpu.sync_copy(ones_v, deg_sh.at[dall.at[k]], add=True)
            return 0

        lax.fori_loop(0, NCH, body, 0)
        plsc.subcore_barrier()
        pltpu.sync_copy(deg_sh.at[pl.ds(sid * stripe, stripe)], zv)
        pltpu.sync_copy(zv, out_hbm.at[pl.ds(cid * R + sid * stripe, stripe)])

    return deg_kernel


def _make_scatter_kernel(H, R, EP, NCH):
    @functools.partial(
        pl.kernel,
        out_type=jax.ShapeDtypeStruct((NC, R, H), jnp.float32),
        mesh=_sc_mesh(),
        scratch_types=[
            pltpu.VMEM((NCH, C), jnp.int32),
            pltpu.VMEM((NCH, C), jnp.int32),
            pltpu.VMEM((C, H), jnp.float32),
            pltpu.VMEM((L, H), jnp.float32),
            pltpu.VMEM_SHARED((R, H), jnp.float32),
        ],
    )
    def scat_kernel(g_hbm, srck_hbm, dstk_hbm, out_hbm,
                    sall, dall, rb, zb, acc_sh):
        cid = lax.axis_index("c")
        sid = lax.axis_index("s")
        wid = sid * NC + cid
        stripe = R // NS

        zero16 = jnp.zeros((L,), jnp.float32)
        for i in range(L):
            for j in range(H // L):
                zb[i, pl.ds(j * L, L)] = zero16

        def zbody(r, _):
            pltpu.sync_copy(zb, acc_sh.at[pl.ds(sid * stripe + r * L, L), :])
            return 0

        lax.fori_loop(0, stripe // L, zbody, 0)
        pltpu.sync_copy(srck_hbm.at[wid], sall)
        pltpu.sync_copy(dstk_hbm.at[wid], dall)
        plsc.subcore_barrier()

        gsrc = g_hbm.at[1 - cid]

        def body(k, _):
            pltpu.sync_copy(gsrc.at[sall.at[k]], rb)
            pltpu.sync_copy(rb, acc_sh.at[dall.at[k]], add=True)
            return 0

        lax.fori_loop(0, NCH, body, 0)
        plsc.subcore_barrier()

        def obody(r, _):
            row0 = sid * stripe + r * C
            pltpu.sync_copy(acc_sh.at[pl.ds(row0, C), :], rb)
            pltpu.sync_copy(rb, out_hbm.at[cid, pl.ds(row0, C), :])
            return 0

        lax.fori_loop(0, stripe // C, obody, 0)

    return scat_kernel


def _tcab_body(x_ref, gamma_ref, beta_ref, w_ref, deg_ref, g_ref, dinv_ref):
    x = x_ref[...]
    n = x.shape[0]
    mean = jnp.mean(x, axis=0, keepdims=True)
    xc = x - mean
    var = jnp.mean(xc * xc, axis=0, keepdims=True)
    xn = gamma_ref[...] * xc * lax.rsqrt(var + 1e-5) + beta_ref[...]
    deg = deg_ref[0, :n, :] + deg_ref[1, :n, :] + 1.0
    dinv = lax.rsqrt(deg)
    dinv_ref[...] = dinv
    g = jnp.dot(xn * dinv, w_ref[...], preferred_element_type=jnp.float32)
    g_ref[0] = g
    g_ref[1] = g


def _tcc_body(acc_ref, g_ref, dinv_ref, b_ref, out_ref):
    s = acc_ref[0] + acc_ref[1] + g_ref[0]
    out_ref[...] = jnp.maximum(dinv_ref[...] * s + b_ref[...], 0.0)


def kernel(x, edge_index, gamma, beta, W, b):
    N, D = x.shape
    H = W.shape[1]
    E = edge_index.shape[1]

    R = ((N + 1 + NS * C - 1) // (NS * C)) * (NS * C)
    Ep = ((E + 4 * NW * C - 1) // (4 * NW * C)) * (4 * NW * C)
    EP = Ep // NW
    NCH = EP // C

    src = edge_index[0]
    dst = edge_index[1]
    pad = Ep - E
    if pad:
        src = jnp.concatenate([src, jnp.zeros((pad,), jnp.int32)])
        dst = jnp.concatenate(
            [dst, N + (jnp.arange(pad, dtype=jnp.int32) % (R - N))])

    dstk = dst.reshape(NW, NCH, C)
    srck = src.reshape(NW, NCH, C)

    deg2 = _make_deg_kernel(R, NCH)(dstk)
    deg_col = deg2.reshape(NC, R)[:, :, None]

    g, dinv = pl.pallas_call(
        _tcab_body,
        out_shape=(
            jax.ShapeDtypeStruct((2, N, H), jnp.float32),
            jax.ShapeDtypeStruct((N, 1), jnp.float32),
        ),
    )(x, gamma, beta, W, deg_col)

    acc2 = _make_scatter_kernel(H, R, EP, NCH)(g, srck, dstk)

    BN = 1000
    grid = (N // BN,)
    out = pl.pallas_call(
        _tcc_body,
        grid=grid,
        in_specs=[
            pl.BlockSpec((NC, BN, H), lambda i: (0, i, 0)),
            pl.BlockSpec((1, BN, H), lambda i: (0, i, 0)),
            pl.BlockSpec((BN, 1), lambda i: (i, 0)),
            pl.BlockSpec((H,), lambda i: (0,)),
        ],
        out_specs=pl.BlockSpec((BN, H), lambda i: (i, 0)),
        out_shape=jax.ShapeDtypeStruct((N, H), jnp.float32),
    )(acc2, g, dinv, b)
    return out


# speedup vs baseline: 1.3483x; 1.3483x over previous
import functools

import jax
import jax.numpy as jnp
from jax import lax
from jax.experimental import pallas as pl
from jax.experimental.pallas import tpu as pltpu
from jax.experimental.pallas import tpu_sc as plsc

NC = 2
NS = 16
NW = NC * NS
C = 128
L = 16
NB = 2


def _sc_mesh():
    return plsc.VectorSubcoreMesh(core_axis_name="c", subcore_axis_name="s")


def _make_deg_kernel(R, NCH):
    @functools.partial(
        pl.kernel,
        out_type=jax.ShapeDtypeStruct((NC * R,), jnp.float32),
        mesh=_sc_mesh(),
        scratch_types=[
            pltpu.VMEM((NCH, C), jnp.int32),
            pltpu.VMEM((C,), jnp.float32),
            pltpu.VMEM((R // NS,), jnp.float32),
            pltpu.VMEM_SHARED((R,), jnp.float32),
        ],
    )
    def deg_kernel(dst_hbm, out_hbm, dall, ones_v, zv, deg_sh):
        cid = lax.axis_index("c")
        sid = lax.axis_index("s")
        wid = sid * NC + cid
        stripe = R // NS

        ones16 = jnp.ones((L,), jnp.float32)
        zero16 = jnp.zeros((L,), jnp.float32)
        for j in range(C // L):
            ones_v[pl.ds(j * L, L)] = ones16
        for j in range(stripe // L):
            zv[pl.ds(j * L, L)] = zero16
        pltpu.sync_copy(dst_hbm.at[wid], dall)
        pltpu.sync_copy(zv, deg_sh.at[pl.ds(sid * stripe, stripe)])
        plsc.subcore_barrier()

        def body(k, _):
            pltpu.sync_copy(ones_v, deg_sh.at[dall.at[k]], add=True)
            return 0

        lax.fori_loop(0, NCH, body, 0)
        plsc.subcore_barrier()
        pltpu.sync_copy(deg_sh.at[pl.ds(sid * stripe, stripe)], zv)
        pltpu.sync_copy(zv, out_hbm.at[pl.ds(cid * R + sid * stripe, stripe)])

    return deg_kernel


def _make_scatter_kernel(H, R, EP, NCH):
    assert NCH % NB == 0 and NCH // NB >= 2

    @functools.partial(
        pl.kernel,
        out_type=jax.ShapeDtypeStruct((NC, R, H), jnp.float32),
        mesh=_sc_mesh(),
        scratch_types=(
            [pltpu.VMEM((NCH, C), jnp.int32)]
            + [pltpu.VMEM((C,), jnp.int32)] * NB
            + [pltpu.VMEM((C, H), jnp.float32)] * NB
            + [pltpu.VMEM((L, H), jnp.float32),
               pltpu.VMEM_SHARED((R, H), jnp.float32)]
            + [pltpu.SemaphoreType.DMA] * (3 * NB)
        ),
    )
    def scat_kernel(g_hbm, srck_hbm, dst_hbm, out_hbm,
                    sall, db0, db1, rb0, rb1, zb, acc_sh, *sems):
        rbs = (rb0, rb1)
        dbs = (db0, db1)
        gsem = sems[:NB]
        ssem = sems[NB:2 * NB]
        dsem = sems[2 * NB:]
        cid = lax.axis_index("c")
        sid = lax.axis_index("s")
        wid = sid * NC + cid
        base = wid * EP
        stripe = R // NS

        zero16 = jnp.zeros((L,), jnp.float32)
        for i in range(L):
            for j in range(H // L):
                zb[i, pl.ds(j * L, L)] = zero16

        def zbody(r, _):
            pltpu.sync_copy(zb, acc_sh.at[pl.ds(sid * stripe + r * L, L), :])
            return 0

        lax.fori_loop(0, stripe // L, zbody, 0)
        pltpu.sync_copy(srck_hbm.at[wid], sall)
        plsc.subcore_barrier()

        gsrc = g_hbm.at[cid]

        def dstart(k, j):
            pltpu.async_copy(dst_hbm.at[pl.ds(base + k * C, C)], dbs[j],
                             dsem[j])

        def dwait(j):
            pltpu.make_async_copy(dst_hbm.at[pl.ds(0, C)], dbs[j],
                                  dsem[j]).wait()

        def gstart(k, j):
            pltpu.async_copy(gsrc.at[sall.at[k]], rbs[j], gsem[j])

        def gwait(j):
            pltpu.make_async_copy(gsrc.at[pl.ds(0, C)], rbs[j], gsem[j]).wait()

        def sstart(j):
            pltpu.async_copy(rbs[j], acc_sh.at[dbs[j]], ssem[j], add=True)

        def swait(j):
            pltpu.make_async_copy(rbs[j], acc_sh.at[pl.ds(0, C)],
                                  ssem[j]).wait()

        for j in range(NB):
            dstart(j, j)
            gstart(j, j)

        def body(m, _):
            a = m * NB
            for j in range(NB):
                gwait(j)
                dwait(j)
                sstart(j)
            for j in range(NB):
                swait(j)
                dstart(a + NB + j, j)
                gstart(a + NB + j, j)
            return 0

        lax.fori_loop(0, NCH // NB - 1, body, 0)
        for j in range(NB):
            gwait(j)
            dwait(j)
            sstart(j)
        for j in range(NB):
            swait(j)
        plsc.subcore_barrier()

        def obody(r, _):
            row0 = sid * stripe + r * C
            pltpu.sync_copy(acc_sh.at[pl.ds(row0, C), :], rb0)
            pltpu.sync_copy(rb0, out_hbm.at[cid, pl.ds(row0, C), :])
            return 0

        lax.fori_loop(0, stripe // C, obody, 0)

    return scat_kernel


def _tcab_body(x_ref, gamma_ref, beta_ref, w_ref, deg_ref, g_ref, dinv_ref):
    x = x_ref[...]
    n = x.shape[0]
    mean = jnp.mean(x, axis=0, keepdims=True)
    xc = x - mean
    var = jnp.mean(xc * xc, axis=0, keepdims=True)
    xn = gamma_ref[...] * xc * lax.rsqrt(var + 1e-5) + beta_ref[...]
    deg = deg_ref[0, :n, :] + deg_ref[1, :n, :] + 1.0
    dinv = lax.rsqrt(deg)
    dinv_ref[...] = dinv
    g = jnp.dot(xn * dinv, w_ref[...], preferred_element_type=jnp.float32)
    g_ref[0] = g
    g_ref[1] = g


def _tcc_body(acc_ref, g_ref, dinv_ref, b_ref, out_ref):
    s = acc_ref[0] + acc_ref[1] + g_ref[0]
    out_ref[...] = jnp.maximum(dinv_ref[...] * s + b_ref[...], 0.0)


def kernel(x, edge_index, gamma, beta, W, b):
    N, D = x.shape
    H = W.shape[1]
    E = edge_index.shape[1]

    R = ((N + 1 + NS * C - 1) // (NS * C)) * (NS * C)
    Ep = ((E + NB * NW * C - 1) // (NB * NW * C)) * (NB * NW * C)
    EP = Ep // NW
    NCH = EP // C

    src = edge_index[0]
    dst = edge_index[1]
    pad = Ep - E
    if pad:
        src = jnp.concatenate([src, jnp.zeros((pad,), jnp.int32)])
        dst = jnp.concatenate(
            [dst, N + (jnp.arange(pad, dtype=jnp.int32) % (R - N))])

    dstk = dst.reshape(NW, NCH, C)
    srck = src.reshape(NW, NCH, C)

    deg2 = _make_deg_kernel(R, NCH)(dstk)
    deg_col = deg2.reshape(NC, R)[:, :, None]

    g, dinv = pl.pallas_call(
        _tcab_body,
        out_shape=(
            jax.ShapeDtypeStruct((2, N, H), jnp.float32),
            jax.ShapeDtypeStruct((N, 1), jnp.float32),
        ),
    )(x, gamma, beta, W, deg_col)

    acc2 = _make_scatter_kernel(H, R, EP, NCH)(g, srck, dst)

    BN = 1000
    grid = (N // BN,)
    out = pl.pallas_call(
        _tcc_body,
        grid=grid,
        in_specs=[
            pl.BlockSpec((NC, BN, H), lambda i: (0, i, 0)),
            pl.BlockSpec((1, BN, H), lambda i: (0, i, 0)),
            pl.BlockSpec((BN, 1), lambda i: (i, 0)),
            pl.BlockSpec((H,), lambda i: (0,)),
        ],
        out_specs=pl.BlockSpec((BN, H), lambda i: (i, 0)),
        out_shape=jax.ShapeDtypeStruct((N, H), jnp.float32),
    )(acc2, g, dinv, b)
    return out


# speedup vs baseline: 1.4142x; 1.0489x over previous
import functools

import jax
import jax.numpy as jnp
from jax import lax
from jax.experimental import pallas as pl
from jax.experimental.pallas import tpu as pltpu
from jax.experimental.pallas import tpu_sc as plsc

NC = 2
NS = 16
NW = NC * NS
C = 128
L = 16
NB = 2


def _sc_mesh():
    return plsc.VectorSubcoreMesh(core_axis_name="c", subcore_axis_name="s")


def _make_deg_kernel(R, NCH):
    @functools.partial(
        pl.kernel,
        out_type=jax.ShapeDtypeStruct((NC * R,), jnp.float32),
        mesh=_sc_mesh(),
        scratch_types=[
            pltpu.VMEM((NCH, C), jnp.int32),
            pltpu.VMEM((C,), jnp.float32),
            pltpu.VMEM((R // NS,), jnp.float32),
            pltpu.VMEM_SHARED((R,), jnp.float32),
        ],
    )
    def deg_kernel(dst_hbm, out_hbm, dall, ones_v, zv, deg_sh):
        cid = lax.axis_index("c")
        sid = lax.axis_index("s")
        wid = sid * NC + cid
        stripe = R // NS

        ones16 = jnp.ones((L,), jnp.float32)
        zero16 = jnp.zeros((L,), jnp.float32)
        for j in range(C // L):
            ones_v[pl.ds(j * L, L)] = ones16
        for j in range(stripe // L):
            zv[pl.ds(j * L, L)] = zero16
        pltpu.sync_copy(dst_hbm.at[wid], dall)
        pltpu.sync_copy(zv, deg_sh.at[pl.ds(sid * stripe, stripe)])
        plsc.subcore_barrier()

        def body(k, _):
            pltpu.sync_copy(ones_v, deg_sh.at[dall.at[k]], add=True)
            return 0

        lax.fori_loop(0, NCH, body, 0)
        plsc.subcore_barrier()
        pltpu.sync_copy(deg_sh.at[pl.ds(sid * stripe, stripe)], zv)
        pltpu.sync_copy(zv, out_hbm.at[pl.ds(cid * R + sid * stripe, stripe)])

    return deg_kernel


def _make_scatter_kernel(H, R, NCH0, NCH1):
    assert NCH0 % NB == 0 and NCH1 % NB == 0 and NCH1 // NB >= 2
    assert NCH0 >= NCH1

    @functools.partial(
        pl.kernel,
        out_type=jax.ShapeDtypeStruct((NC, R, H), jnp.float32),
        mesh=_sc_mesh(),
        scratch_types=(
            [pltpu.VMEM((NCH0, C), jnp.int32)]
            + [pltpu.VMEM((C,), jnp.int32)] * NB
            + [pltpu.VMEM((C, H), jnp.float32)] * NB
            + [pltpu.VMEM_SHARED((R, H), jnp.float32)]
            + [pltpu.SemaphoreType.DMA] * (3 * NB)
        ),
    )
    def scat_kernel(g_hbm, srck_hbm, dst_hbm, out_hbm,
                    sall, db0, db1, rb0, rb1, acc_sh, *sems):
        rbs = (rb0, rb1)
        dbs = (db0, db1)
        gsem = sems[:NB]
        ssem = sems[NB:2 * NB]
        dsem = sems[2 * NB:]
        cid = lax.axis_index("c")
        sid = lax.axis_index("s")
        stripe = R // NS
        chunk_base = jnp.where(cid == 0, sid * NCH0, NS * NCH0 + sid * NCH1)
        nch = jnp.where(cid == 0, NCH0, NCH1)
        base = chunk_base * C

        zero16 = jnp.zeros((L,), jnp.float32)
        for i in range(L):
            for j in range(H // L):
                rb0[i, pl.ds(j * L, L)] = zero16

        def zbody(r, _):
            pltpu.sync_copy(rb0.at[pl.ds(0, L), :],
                            acc_sh.at[pl.ds(sid * stripe + r * L, L), :])
            return 0

        lax.fori_loop(0, stripe // L, zbody, 0)
        pltpu.sync_copy(srck_hbm.at[pl.ds(chunk_base, NCH0)], sall)
        plsc.subcore_barrier()

        gsrc = g_hbm.at[cid]

        def dstart(k, j):
            pltpu.async_copy(dst_hbm.at[pl.ds(base + k * C, C)], dbs[j],
                             dsem[j])

        def dwait(j):
            pltpu.make_async_copy(dst_hbm.at[pl.ds(0, C)], dbs[j],
                                  dsem[j]).wait()

        def gstart(k, j):
            pltpu.async_copy(gsrc.at[sall.at[k]], rbs[j], gsem[j])

        def gwait(j):
            pltpu.make_async_copy(gsrc.at[pl.ds(0, C)], rbs[j], gsem[j]).wait()

        def sstart(j):
            pltpu.async_copy(rbs[j], acc_sh.at[dbs[j]], ssem[j], add=True)

        def swait(j):
            pltpu.make_async_copy(rbs[j], acc_sh.at[pl.ds(0, C)],
                                  ssem[j]).wait()

        for j in range(NB):
            dstart(j, j)
            gstart(j, j)

        def body(m, _):
            a = m * NB
            for j in range(NB):
                gwait(j)
                dwait(j)
                sstart(j)
            for j in range(NB):
                swait(j)
                dstart(a + NB + j, j)
                gstart(a + NB + j, j)
            return 0

        lax.fori_loop(0, nch // NB - 1, body, 0)
        for j in range(NB):
            gwait(j)
            dwait(j)
            sstart(j)
        for j in range(NB):
            swait(j)
        plsc.subcore_barrier()

        def obody(r, _):
            row0 = sid * stripe + r * C
            pltpu.sync_copy(acc_sh.at[pl.ds(row0, C), :], rb0)
            pltpu.sync_copy(rb0, out_hbm.at[cid, pl.ds(row0, C), :])
            return 0

        lax.fori_loop(0, stripe // C, obody, 0)

    return scat_kernel


def _tcab_body(x_ref, gamma_ref, beta_ref, w_ref, deg_ref, g_ref, dinv_ref):
    x = x_ref[...]
    n = x.shape[0]
    mean = jnp.mean(x, axis=0, keepdims=True)
    xc = x - mean
    var = jnp.mean(xc * xc, axis=0, keepdims=True)
    xn = gamma_ref[...] * xc * lax.rsqrt(var + 1e-5) + beta_ref[...]
    deg = deg_ref[0, :n, :] + deg_ref[1, :n, :] + 1.0
    dinv = lax.rsqrt(deg)
    dinv_ref[...] = dinv
    g = jnp.dot(xn * dinv, w_ref[...], preferred_element_type=jnp.float32)
    g_ref[0] = g
    g_ref[1] = g


def _tcc_body(acc_ref, g_ref, dinv_ref, b_ref, out_ref):
    s = acc_ref[0] + acc_ref[1] + g_ref[0]
    out_ref[...] = jnp.maximum(dinv_ref[...] * s + b_ref[...], 0.0)


def kernel(x, edge_index, gamma, beta, W, b):
    N, D = x.shape
    H = W.shape[1]
    E = edge_index.shape[1]

    R = ((N + 1 + NS * C - 1) // (NS * C)) * (NS * C)
    T = (((E + NS * C - 1) // (NS * C)) + 15) // 16 * 16
    NCH0 = max(8, int(round(0.75 * T / 8)) * 8)
    NCH1 = T - NCH0
    assert NCH1 >= 2 * NB and NCH0 % NB == 0 and NCH1 % NB == 0
    Ep = NS * T * C

    src = edge_index[0]
    dst = edge_index[1]
    pad = Ep - E
    if pad:
        src = jnp.concatenate([src, jnp.zeros((pad,), jnp.int32)])
        dst = jnp.concatenate(
            [dst, N + (jnp.arange(pad, dtype=jnp.int32) % (R - N))])

    srck = jnp.concatenate(
        [src.reshape(NS * T, C),
         jnp.zeros((NCH0 - NCH1, C), jnp.int32)])

    NCHD = Ep // (NW * C)
    dstk = dst.reshape(NW, NCHD, C)
    deg2 = _make_deg_kernel(R, NCHD)(dstk)
    deg_col = deg2.reshape(NC, R)[:, :, None]

    g, dinv = pl.pallas_call(
        _tcab_body,
        out_shape=(
            jax.ShapeDtypeStruct((2, N, H), jnp.float32),
            jax.ShapeDtypeStruct((N, 1), jnp.float32),
        ),
    )(x, gamma, beta, W, deg_col)

    acc2 = _make_scatter_kernel(H, R, NCH0, NCH1)(g, srck, dst)

    BN = 1000
    grid = (N // BN,)
    out = pl.pallas_call(
        _tcc_body,
        grid=grid,
        in_specs=[
            pl.BlockSpec((NC, BN, H), lambda i: (0, i, 0)),
            pl.BlockSpec((1, BN, H), lambda i: (0, i, 0)),
            pl.BlockSpec((BN, 1), lambda i: (i, 0)),
            pl.BlockSpec((H,), lambda i: (0,)),
        ],
        out_specs=pl.BlockSpec((BN, H), lambda i: (i, 0)),
        out_shape=jax.ShapeDtypeStruct((N, H), jnp.float32),
    )(acc2, g, dinv, b)
    return out


# speedup vs baseline: 1.4177x; 1.0024x over previous
import functools

import jax
import jax.numpy as jnp
from jax import lax
from jax.experimental import pallas as pl
from jax.experimental.pallas import tpu as pltpu
from jax.experimental.pallas import tpu_sc as plsc

NC = 2
NS = 16
NW = NC * NS
C = 128
L = 16
NB = 2


def _sc_mesh():
    return plsc.VectorSubcoreMesh(core_axis_name="c", subcore_axis_name="s")


def _make_deg_kernel(R, NCH):
    @functools.partial(
        pl.kernel,
        out_type=jax.ShapeDtypeStruct((NC * R,), jnp.float32),
        mesh=_sc_mesh(),
        scratch_types=[
            pltpu.VMEM((NCH, C), jnp.int32),
            pltpu.VMEM((C,), jnp.float32),
            pltpu.VMEM((R // NS,), jnp.float32),
            pltpu.VMEM_SHARED((R,), jnp.float32),
        ],
    )
    def deg_kernel(dst_hbm, out_hbm, dall, ones_v, zv, deg_sh):
        cid = lax.axis_index("c")
        sid = lax.axis_index("s")
        wid = sid * NC + cid
        stripe = R // NS

        ones16 = jnp.ones((L,), jnp.float32)
        zero16 = jnp.zeros((L,), jnp.float32)
        for j in range(C // L):
            ones_v[pl.ds(j * L, L)] = ones16
        for j in range(stripe // L):
            zv[pl.ds(j * L, L)] = zero16
        pltpu.sync_copy(dst_hbm.at[wid], dall)
        pltpu.sync_copy(zv, deg_sh.at[pl.ds(sid * stripe, stripe)])
        plsc.subcore_barrier()

        def body(k, _):
            pltpu.sync_copy(ones_v, deg_sh.at[dall.at[k]], add=True)
            return 0

        lax.fori_loop(0, NCH, body, 0)
        plsc.subcore_barrier()
        pltpu.sync_copy(deg_sh.at[pl.ds(sid * stripe, stripe)], zv)
        pltpu.sync_copy(zv, out_hbm.at[pl.ds(cid * R + sid * stripe, stripe)])

    return deg_kernel


def _make_scatter_kernel(H, R, NCH0, NCH1):
    assert NCH0 % NB == 0 and NCH1 % NB == 0 and NCH1 // NB >= 2
    assert NCH0 >= NCH1

    @functools.partial(
        pl.kernel,
        out_type=jax.ShapeDtypeStruct((NC, R, H), jnp.float32),
        mesh=_sc_mesh(),
        scratch_types=(
            [pltpu.VMEM((C,), jnp.int32)] * NB
            + [pltpu.VMEM((C,), jnp.int32)] * NB
            + [pltpu.VMEM((C, H), jnp.float32)] * NB
            + [pltpu.VMEM_SHARED((R, H), jnp.float32)]
            + [pltpu.SemaphoreType.DMA] * (4 * NB)
        ),
    )
    def scat_kernel(g_hbm, src_hbm, dst_hbm, out_hbm,
                    sb0, sb1, db0, db1, rb0, rb1, acc_sh, *sems):
        rbs = (rb0, rb1)
        dbs = (db0, db1)
        sbs = (sb0, sb1)
        gsem = sems[:NB]
        ssem = sems[NB:2 * NB]
        dsem = sems[2 * NB:3 * NB]
        isem = sems[3 * NB:]
        cid = lax.axis_index("c")
        sid = lax.axis_index("s")
        stripe = R // NS
        chunk_base = jnp.where(cid == 0, sid * NCH0, NS * NCH0 + sid * NCH1)
        nch = jnp.where(cid == 0, NCH0, NCH1)
        base = chunk_base * C

        def istart(k, j):
            pltpu.async_copy(src_hbm.at[pl.ds(base + k * C, C)], sbs[j],
                             isem[j])

        def iwait(j):
            pltpu.make_async_copy(src_hbm.at[pl.ds(0, C)], sbs[j],
                                  isem[j]).wait()

        def dstart(k, j):
            pltpu.async_copy(dst_hbm.at[pl.ds(base + k * C, C)], dbs[j],
                             dsem[j])

        def dwait(j):
            pltpu.make_async_copy(dst_hbm.at[pl.ds(0, C)], dbs[j],
                                  dsem[j]).wait()

        gsrc = g_hbm.at[cid]

        def gstart(j):
            pltpu.async_copy(gsrc.at[sbs[j]], rbs[j], gsem[j])

        def gwait(j):
            pltpu.make_async_copy(gsrc.at[pl.ds(0, C)], rbs[j], gsem[j]).wait()

        def sstart(j):
            pltpu.async_copy(rbs[j], acc_sh.at[dbs[j]], ssem[j], add=True)

        def swait(j):
            pltpu.make_async_copy(rbs[j], acc_sh.at[pl.ds(0, C)],
                                  ssem[j]).wait()

        for j in range(NB):
            istart(j, j)
            dstart(j, j)

        zero16 = jnp.zeros((L,), jnp.float32)
        for i in range(L):
            for j in range(H // L):
                rb0[i, pl.ds(j * L, L)] = zero16

        def zbody(r, _):
            pltpu.sync_copy(rb0.at[pl.ds(0, L), :],
                            acc_sh.at[pl.ds(sid * stripe + r * L, L), :])
            return 0

        lax.fori_loop(0, stripe // L, zbody, 0)
        plsc.subcore_barrier()

        for j in range(NB):
            iwait(j)
            gstart(j)

        def body(m, _):
            a = m * NB
            for j in range(NB):
                gwait(j)
                dwait(j)
                sstart(j)
            for j in range(NB):
                swait(j)
                istart(a + NB + j, j)
                dstart(a + NB + j, j)
            for j in range(NB):
                iwait(j)
                gstart(j)
            return 0

        lax.fori_loop(0, nch // NB - 1, body, 0)
        for j in range(NB):
            gwait(j)
            dwait(j)
            sstart(j)
        for j in range(NB):
            swait(j)
        plsc.subcore_barrier()

        def obody(r, _):
            row0 = sid * stripe + r * C
            pltpu.sync_copy(acc_sh.at[pl.ds(row0, C), :], rb0)
            pltpu.sync_copy(rb0, out_hbm.at[cid, pl.ds(row0, C), :])
            return 0

        lax.fori_loop(0, stripe // C, obody, 0)

    return scat_kernel


def _tcab_body(x_ref, gamma_ref, beta_ref, w_ref, deg_ref, g_ref, dinv_ref):
    x = x_ref[...]
    n = x.shape[0]
    mean = jnp.mean(x, axis=0, keepdims=True)
    xc = x - mean
    var = jnp.mean(xc * xc, axis=0, keepdims=True)
    xn = gamma_ref[...] * xc * lax.rsqrt(var + 1e-5) + beta_ref[...]
    deg = deg_ref[0, :n, :] + deg_ref[1, :n, :] + 1.0
    dinv = lax.rsqrt(deg)
    dinv_ref[...] = dinv
    g = jnp.dot(xn * dinv, w_ref[...], preferred_element_type=jnp.float32)
    g_ref[0] = g
    g_ref[1] = g


def _tcc_body(acc_ref, g_ref, dinv_ref, b_ref, out_ref):
    s = acc_ref[0] + acc_ref[1] + g_ref[0]
    out_ref[...] = jnp.maximum(dinv_ref[...] * s + b_ref[...], 0.0)


def kernel(x, edge_index, gamma, beta, W, b):
    N, D = x.shape
    H = W.shape[1]
    E = edge_index.shape[1]

    R = ((N + 1 + NS * C - 1) // (NS * C)) * (NS * C)
    T = (((E + NS * C - 1) // (NS * C)) + 15) // 16 * 16
    NCH0 = max(NB, int(round(0.85 * T / NB)) * NB)
    NCH1 = T - NCH0
    assert NCH1 >= 2 * NB and NCH0 % NB == 0 and NCH1 % NB == 0
    Ep = NS * T * C

    src = edge_index[0]
    dst = edge_index[1]
    pad = Ep - E
    if pad:
        src = jnp.concatenate([src, jnp.zeros((pad,), jnp.int32)])
        dst = jnp.concatenate(
            [dst, N + (jnp.arange(pad, dtype=jnp.int32) % (R - N))])

    NCHD = Ep // (NW * C)
    dstk = dst.reshape(NW, NCHD, C)
    deg2 = _make_deg_kernel(R, NCHD)(dstk)
    deg_col = deg2.reshape(NC, R)[:, :, None]

    g, dinv = pl.pallas_call(
        _tcab_body,
        out_shape=(
            jax.ShapeDtypeStruct((2, N, H), jnp.float32),
            jax.ShapeDtypeStruct((N, 1), jnp.float32),
        ),
    )(x, gamma, beta, W, deg_col)

    acc2 = _make_scatter_kernel(H, R, NCH0, NCH1)(g, src, dst)

    BN = 1000
    grid = (N // BN,)
    out = pl.pallas_call(
        _tcc_body,
        grid=grid,
        in_specs=[
            pl.BlockSpec((NC, BN, H), lambda i: (0, i, 0)),
            pl.BlockSpec((1, BN, H), lambda i: (0, i, 0)),
            pl.BlockSpec((BN, 1), lambda i: (i, 0)),
            pl.BlockSpec((H,), lambda i: (0,)),
        ],
        out_specs=pl.BlockSpec((BN, H), lambda i: (i, 0)),
        out_shape=jax.ShapeDtypeStruct((N, H), jnp.float32),
    )(acc2, g, dinv, b)
    return out


# speedup vs baseline: 1.4197x; 1.0014x over previous
import functools

import jax
import jax.numpy as jnp
from jax import lax
from jax.experimental import pallas as pl
from jax.experimental.pallas import tpu as pltpu
from jax.experimental.pallas import tpu_sc as plsc

NC = 2
NS = 16
NW = NC * NS
C = 128
L = 16
NB = 2
NI = 4


def _sc_mesh():
    return plsc.VectorSubcoreMesh(core_axis_name="c", subcore_axis_name="s")


def _make_deg_kernel(R, NCH):
    @functools.partial(
        pl.kernel,
        out_type=jax.ShapeDtypeStruct((NC * R,), jnp.float32),
        mesh=_sc_mesh(),
        scratch_types=[
            pltpu.VMEM((NCH, C), jnp.int32),
            pltpu.VMEM((C,), jnp.float32),
            pltpu.VMEM((R // NS,), jnp.float32),
            pltpu.VMEM_SHARED((R,), jnp.float32),
        ],
    )
    def deg_kernel(dst_hbm, out_hbm, dall, ones_v, zv, deg_sh):
        cid = lax.axis_index("c")
        sid = lax.axis_index("s")
        wid = sid * NC + cid
        stripe = R // NS

        ones16 = jnp.ones((L,), jnp.float32)
        zero16 = jnp.zeros((L,), jnp.float32)
        for j in range(C // L):
            ones_v[pl.ds(j * L, L)] = ones16
        for j in range(stripe // L):
            zv[pl.ds(j * L, L)] = zero16
        pltpu.sync_copy(dst_hbm.at[wid], dall)
        pltpu.sync_copy(zv, deg_sh.at[pl.ds(sid * stripe, stripe)])
        plsc.subcore_barrier()

        def body(k, _):
            pltpu.sync_copy(ones_v, deg_sh.at[dall.at[k]], add=True)
            return 0

        lax.fori_loop(0, NCH, body, 0)
        plsc.subcore_barrier()
        pltpu.sync_copy(deg_sh.at[pl.ds(sid * stripe, stripe)], zv)
        pltpu.sync_copy(zv, out_hbm.at[pl.ds(cid * R + sid * stripe, stripe)])

    return deg_kernel


def _make_scatter_kernel(H, R, NCH0, NCH1):
    assert NCH0 % NI == 0 and NCH1 % NI == 0 and NCH1 // NI >= 2
    assert NCH0 >= NCH1 and NI == 2 * NB

    @functools.partial(
        pl.kernel,
        out_type=jax.ShapeDtypeStruct((NC, R, H), jnp.float32),
        mesh=_sc_mesh(),
        scratch_types=(
            [pltpu.VMEM((C,), jnp.int32)] * NI
            + [pltpu.VMEM((C,), jnp.int32)] * NI
            + [pltpu.VMEM((C, H), jnp.float32)] * NB
            + [pltpu.VMEM_SHARED((R, H), jnp.float32)]
            + [pltpu.SemaphoreType.DMA] * (2 * NI + 2 * NB)
        ),
    )
    def scat_kernel(g_hbm, src_hbm, dst_hbm, out_hbm,
                    sb0, sb1, sb2, sb3, db0, db1, db2, db3,
                    rb0, rb1, acc_sh, *sems):
        rbs = (rb0, rb1)
        dbs = (db0, db1, db2, db3)
        sbs = (sb0, sb1, sb2, sb3)
        isem = sems[:NI]
        dsem = sems[NI:2 * NI]
        gsem = sems[2 * NI:2 * NI + NB]
        ssem = sems[2 * NI + NB:]
        cid = lax.axis_index("c")
        sid = lax.axis_index("s")
        stripe = R // NS
        chunk_base = jnp.where(cid == 0, sid * NCH0, NS * NCH0 + sid * NCH1)
        nch = jnp.where(cid == 0, NCH0, NCH1)
        base = chunk_base * C

        def istart(k, j):
            pltpu.async_copy(src_hbm.at[pl.ds(base + k * C, C)], sbs[j],
                             isem[j])

        def iwait(j):
            pltpu.make_async_copy(src_hbm.at[pl.ds(0, C)], sbs[j],
                                  isem[j]).wait()

        def dstart(k, j):
            pltpu.async_copy(dst_hbm.at[pl.ds(base + k * C, C)], dbs[j],
                             dsem[j])

        def dwait(j):
            pltpu.make_async_copy(dst_hbm.at[pl.ds(0, C)], dbs[j],
                                  dsem[j]).wait()

        gsrc = g_hbm.at[cid]

        def gstart(r, j):
            pltpu.async_copy(gsrc.at[sbs[j]], rbs[r], gsem[r])

        def gwait(r):
            pltpu.make_async_copy(gsrc.at[pl.ds(0, C)], rbs[r], gsem[r]).wait()

        def sstart(r, j):
            pltpu.async_copy(rbs[r], acc_sh.at[dbs[j]], ssem[r], add=True)

        def swait(r):
            pltpu.make_async_copy(rbs[r], acc_sh.at[pl.ds(0, C)],
                                  ssem[r]).wait()

        for j in range(NI):
            istart(j, j)
            dstart(j, j)

        zero16 = jnp.zeros((L,), jnp.float32)
        for i in range(L):
            for j in range(H // L):
                rb0[i, pl.ds(j * L, L)] = zero16

        def zbody(r, _):
            pltpu.sync_copy(rb0.at[pl.ds(0, L), :],
                            acc_sh.at[pl.ds(sid * stripe + r * L, L), :])
            return 0

        lax.fori_loop(0, stripe // L, zbody, 0)
        plsc.subcore_barrier()

        for r in range(NB):
            iwait(r)
            gstart(r, r)

        def body(m, _):
            a = m * NI
            for j in range(NB):
                gwait(j)
                dwait(j)
                sstart(j, j)
            for j in range(NB):
                swait(j)
                istart(a + NI + j, j)
                dstart(a + NI + j, j)
                iwait(NB + j)
                gstart(j, NB + j)
            for j in range(NB):
                gwait(j)
                dwait(NB + j)
                sstart(j, NB + j)
            for j in range(NB):
                swait(j)
                istart(a + NI + NB + j, NB + j)
                dstart(a + NI + NB + j, NB + j)
                iwait(j)
                gstart(j, j)
            return 0

        lax.fori_loop(0, nch // NI - 1, body, 0)

        for j in range(NB):
            gwait(j)
            dwait(j)
            sstart(j, j)
        for j in range(NB):
            swait(j)
            iwait(NB + j)
            gstart(j, NB + j)
        for j in range(NB):
            gwait(j)
            dwait(NB + j)
            sstart(j, NB + j)
        for j in range(NB):
            swait(j)
        plsc.subcore_barrier()

        def obody(r, _):
            row0 = sid * stripe + r * C
            pltpu.sync_copy(acc_sh.at[pl.ds(row0, C), :], rb0)
            pltpu.sync_copy(rb0, out_hbm.at[cid, pl.ds(row0, C), :])
            return 0

        lax.fori_loop(0, stripe // C, obody, 0)

    return scat_kernel


def _tcab_body(x_ref, gamma_ref, beta_ref, w_ref, deg_ref, g_ref, dinv_ref):
    x = x_ref[...]
    n = x.shape[0]
    mean = jnp.mean(x, axis=0, keepdims=True)
    xc = x - mean
    var = jnp.mean(xc * xc, axis=0, keepdims=True)
    xn = gamma_ref[...] * xc * lax.rsqrt(var + 1e-5) + beta_ref[...]
    deg = deg_ref[0, :n, :] + deg_ref[1, :n, :] + 1.0
    dinv = lax.rsqrt(deg)
    dinv_ref[...] = dinv
    g = jnp.dot(xn * dinv, w_ref[...], preferred_element_type=jnp.float32)
    g_ref[0] = g
    g_ref[1] = g


def _tcc_body(acc_ref, g_ref, dinv_ref, b_ref, out_ref):
    s = acc_ref[0] + acc_ref[1] + g_ref[0]
    out_ref[...] = jnp.maximum(dinv_ref[...] * s + b_ref[...], 0.0)


def kernel(x, edge_index, gamma, beta, W, b):
    N, D = x.shape
    H = W.shape[1]
    E = edge_index.shape[1]

    R = ((N + 1 + NS * C - 1) // (NS * C)) * (NS * C)
    T = (((E + NS * C - 1) // (NS * C)) + 15) // 16 * 16
    NCH0 = max(NI, int(round(0.85 * T / NI)) * NI)
    NCH1 = T - NCH0
    assert NCH1 >= 2 * NI and NCH0 % NI == 0 and NCH1 % NI == 0
    Ep = NS * T * C

    src = edge_index[0]
    dst = edge_index[1]
    pad = Ep - E
    if pad:
        src = jnp.concatenate([src, jnp.zeros((pad,), jnp.int32)])
        dst = jnp.concatenate(
            [dst, N + (jnp.arange(pad, dtype=jnp.int32) % (R - N))])

    NCHD = Ep // (NW * C)
    dstk = dst.reshape(NW, NCHD, C)
    deg2 = _make_deg_kernel(R, NCHD)(dstk)
    deg_col = deg2.reshape(NC, R)[:, :, None]

    g, dinv = pl.pallas_call(
        _tcab_body,
        out_shape=(
            jax.ShapeDtypeStruct((2, N, H), jnp.float32),
            jax.ShapeDtypeStruct((N, 1), jnp.float32),
        ),
    )(x, gamma, beta, W, deg_col)

    acc2 = _make_scatter_kernel(H, R, NCH0, NCH1)(g, src, dst)

    BN = 1000
    grid = (N // BN,)
    out = pl.pallas_call(
        _tcc_body,
        grid=grid,
        in_specs=[
            pl.BlockSpec((NC, BN, H), lambda i: (0, i, 0)),
            pl.BlockSpec((1, BN, H), lambda i: (0, i, 0)),
            pl.BlockSpec((BN, 1), lambda i: (i, 0)),
            pl.BlockSpec((H,), lambda i: (0,)),
        ],
        out_specs=pl.BlockSpec((BN, H), lambda i: (i, 0)),
        out_shape=jax.ShapeDtypeStruct((N, H), jnp.float32),
    )(acc2, g, dinv, b)
    return out


# speedup vs baseline: 1.4199x; 1.0002x over previous
import functools

import jax
import jax.numpy as jnp
from jax import lax
from jax.experimental import pallas as pl
from jax.experimental.pallas import tpu as pltpu
from jax.experimental.pallas import tpu_sc as plsc

NC = 2
NS = 16
NW = NC * NS
C = 128
L = 16
NB = 2
B = 4


def _sc_mesh():
    return plsc.VectorSubcoreMesh(core_axis_name="c", subcore_axis_name="s")


def _make_deg_kernel(R, NCH):
    @functools.partial(
        pl.kernel,
        out_type=jax.ShapeDtypeStruct((NC * R,), jnp.float32),
        mesh=_sc_mesh(),
        scratch_types=[
            pltpu.VMEM((NCH, C), jnp.int32),
            pltpu.VMEM((C,), jnp.float32),
            pltpu.VMEM((R // NS,), jnp.float32),
            pltpu.VMEM_SHARED((R,), jnp.float32),
        ],
    )
    def deg_kernel(dst_hbm, out_hbm, dall, ones_v, zv, deg_sh):
        cid = lax.axis_index("c")
        sid = lax.axis_index("s")
        wid = sid * NC + cid
        stripe = R // NS

        ones16 = jnp.ones((L,), jnp.float32)
        zero16 = jnp.zeros((L,), jnp.float32)
        for j in range(C // L):
            ones_v[pl.ds(j * L, L)] = ones16
        for j in range(stripe // L):
            zv[pl.ds(j * L, L)] = zero16
        pltpu.sync_copy(dst_hbm.at[wid], dall)
        pltpu.sync_copy(zv, deg_sh.at[pl.ds(sid * stripe, stripe)])
        plsc.subcore_barrier()

        def body(k, _):
            pltpu.sync_copy(ones_v, deg_sh.at[dall.at[k]], add=True)
            return 0

        lax.fori_loop(0, NCH, body, 0)
        plsc.subcore_barrier()
        pltpu.sync_copy(deg_sh.at[pl.ds(sid * stripe, stripe)], zv)
        pltpu.sync_copy(zv, out_hbm.at[pl.ds(cid * R + sid * stripe, stripe)])

    return deg_kernel


def _make_scatter_kernel(H, R, NCH0, NCH1):
    assert NCH0 % (2 * B) == 0 and NCH1 % (2 * B) == 0 and NCH1 >= 4 * B
    assert NCH0 >= NCH1 and B % NB == 0

    @functools.partial(
        pl.kernel,
        out_type=jax.ShapeDtypeStruct((NC, R, H), jnp.float32),
        mesh=_sc_mesh(),
        scratch_types=(
            [pltpu.VMEM((B * C,), jnp.int32)] * 2
            + [pltpu.VMEM((B * C,), jnp.int32)] * 2
            + [pltpu.VMEM((C, H), jnp.float32)] * NB
            + [pltpu.VMEM_SHARED((R, H), jnp.float32)]
            + [pltpu.SemaphoreType.DMA] * (4 + 2 * NB)
        ),
    )
    def scat_kernel(g_hbm, src_hbm, dst_hbm, out_hbm,
                    sb0, sb1, db0, db1, rb0, rb1, acc_sh, *sems):
        rbs = (rb0, rb1)
        dbs = (db0, db1)
        sbs = (sb0, sb1)
        isem = sems[:2]
        dsem = sems[2:4]
        gsem = sems[4:4 + NB]
        ssem = sems[4 + NB:]
        cid = lax.axis_index("c")
        sid = lax.axis_index("s")
        stripe = R // NS
        chunk_base = jnp.where(cid == 0, sid * NCH0, NS * NCH0 + sid * NCH1)
        nch = jnp.where(cid == 0, NCH0, NCH1)
        base = chunk_base * C

        def istart(k, j):
            pltpu.async_copy(src_hbm.at[pl.ds(base + k * C, B * C)], sbs[j],
                             isem[j])

        def iwait(j):
            pltpu.make_async_copy(src_hbm.at[pl.ds(0, B * C)], sbs[j],
                                  isem[j]).wait()

        def dstart(k, j):
            pltpu.async_copy(dst_hbm.at[pl.ds(base + k * C, B * C)], dbs[j],
                             dsem[j])

        def dwait(j):
            pltpu.make_async_copy(dst_hbm.at[pl.ds(0, B * C)], dbs[j],
                                  dsem[j]).wait()

        gsrc = g_hbm.at[cid]

        def gstart(r, blk, q):
            pltpu.async_copy(gsrc.at[sbs[blk].at[pl.ds(q * C, C)]], rbs[r],
                             gsem[r])

        def gwait(r):
            pltpu.make_async_copy(gsrc.at[pl.ds(0, C)], rbs[r], gsem[r]).wait()

        def sstart(r, blk, q):
            pltpu.async_copy(rbs[r], acc_sh.at[dbs[blk].at[pl.ds(q * C, C)]],
                             ssem[r], add=True)

        def swait(r):
            pltpu.make_async_copy(rbs[r], acc_sh.at[pl.ds(0, C)],
                                  ssem[r]).wait()

        for j in range(2):
            istart(j * B, j)
            dstart(j * B, j)

        zero16 = jnp.zeros((L,), jnp.float32)
        for i in range(L):
            for j in range(H // L):
                rb0[i, pl.ds(j * L, L)] = zero16

        def zbody(r, _):
            pltpu.sync_copy(rb0.at[pl.ds(0, L), :],
                            acc_sh.at[pl.ds(sid * stripe + r * L, L), :])
            return 0

        lax.fori_loop(0, stripe // L, zbody, 0)
        plsc.subcore_barrier()

        iwait(0)
        for r in range(NB):
            gstart(r, 0, r)

        def run_block(blk, refill_k):
            oth = 1 - blk
            dwait(blk)
            for p in range(B // NB):
                for r in range(NB):
                    gwait(r)
                    sstart(r, blk, p * NB + r)
                if p < B // NB - 1:
                    for r in range(NB):
                        swait(r)
                        gstart(r, blk, (p + 1) * NB + r)
                else:
                    iwait(oth)
                    for r in range(NB):
                        swait(r)
                        gstart(r, oth, r)
            if refill_k is not None:
                istart(refill_k, blk)
                dstart(refill_k, blk)

        def body(m, _):
            a = m * 2 * B
            run_block(0, a + 2 * B)
            run_block(1, a + 3 * B)
            return 0

        lax.fori_loop(0, nch // (2 * B) - 1, body, 0)

        run_block(0, None)
        dwait(1)
        for p in range(B // NB):
            for r in range(NB):
                gwait(r)
                sstart(r, 1, p * NB + r)
            if p < B // NB - 1:
                for r in range(NB):
                    swait(r)
                    gstart(r, 1, (p + 1) * NB + r)
        for r in range(NB):
            swait(r)
        plsc.subcore_barrier()

        def obody(r, _):
            row0 = sid * stripe + r * C
            pltpu.sync_copy(acc_sh.at[pl.ds(row0, C), :], rb0)
            pltpu.sync_copy(rb0, out_hbm.at[cid, pl.ds(row0, C), :])
            return 0

        lax.fori_loop(0, stripe // C, obody, 0)

    return scat_kernel


def _tcab_body(x_ref, gamma_ref, beta_ref, w_ref, deg_ref, g_ref, dinv_ref):
    x = x_ref[...]
    n = x.shape[0]
    mean = jnp.mean(x, axis=0, keepdims=True)
    xc = x - mean
    var = jnp.mean(xc * xc, axis=0, keepdims=True)
    xn = gamma_ref[...] * xc * lax.rsqrt(var + 1e-5) + beta_ref[...]
    deg = deg_ref[0, :n, :] + deg_ref[1, :n, :] + 1.0
    dinv = lax.rsqrt(deg)
    dinv_ref[...] = dinv
    g = jnp.dot(xn * dinv, w_ref[...], preferred_element_type=jnp.float32)
    g_ref[0] = g
    g_ref[1] = g


def _tcc_body(acc_ref, g_ref, dinv_ref, b_ref, out_ref):
    s = acc_ref[0] + acc_ref[1] + g_ref[0]
    out_ref[...] = jnp.maximum(dinv_ref[...] * s + b_ref[...], 0.0)


def kernel(x, edge_index, gamma, beta, W, b):
    N, D = x.shape
    H = W.shape[1]
    E = edge_index.shape[1]

    R = ((N + 1 + NS * C - 1) // (NS * C)) * (NS * C)
    T = (((E + NS * C - 1) // (NS * C)) + 15) // 16 * 16
    NCH0 = max(2 * B, int(round(0.85 * T / (2 * B))) * (2 * B))
    NCH1 = T - NCH0
    assert NCH1 >= 4 * B and NCH0 % (2 * B) == 0 and NCH1 % (2 * B) == 0
    Ep = NS * T * C

    src = edge_index[0]
    dst = edge_index[1]
    pad = Ep - E
    if pad:
        src = jnp.concatenate([src, jnp.zeros((pad,), jnp.int32)])
        dst = jnp.concatenate(
            [dst, N + (jnp.arange(pad, dtype=jnp.int32) % (R - N))])

    NCHD = Ep // (NW * C)
    dstk = dst.reshape(NW, NCHD, C)
    deg2 = _make_deg_kernel(R, NCHD)(dstk)
    deg_col = deg2.reshape(NC, R)[:, :, None]

    g, dinv = pl.pallas_call(
        _tcab_body,
        out_shape=(
            jax.ShapeDtypeStruct((2, N, H), jnp.float32),
            jax.ShapeDtypeStruct((N, 1), jnp.float32),
        ),
    )(x, gamma, beta, W, deg_col)

    acc2 = _make_scatter_kernel(H, R, NCH0, NCH1)(g, src, dst)

    BN = 1000
    grid = (N // BN,)
    out = pl.pallas_call(
        _tcc_body,
        grid=grid,
        in_specs=[
            pl.BlockSpec((NC, BN, H), lambda i: (0, i, 0)),
            pl.BlockSpec((1, BN, H), lambda i: (0, i, 0)),
            pl.BlockSpec((BN, 1), lambda i: (i, 0)),
            pl.BlockSpec((H,), lambda i: (0,)),
        ],
        out_specs=pl.BlockSpec((BN, H), lambda i: (i, 0)),
        out_shape=jax.ShapeDtypeStruct((N, H), jnp.float32),
    )(acc2, g, dinv, b)
    return out
